# Initial kernel scaffold; baseline (speedup 1.0000x reference)
import jax
import jax.numpy as jnp
from jax.experimental import pallas as pl


def kernel(x, votes, scales, vote_presence_prob):
    raise NotImplementedError("write your pallas kernel here")


# speedup vs baseline: 5.4957x; 5.4957x over previous
import functools

import jax
import jax.numpy as jnp
from jax import lax
from jax.experimental import pallas as pl
from jax.experimental.pallas import tpu as pltpu


def _capsule_kernel(x_ref, votes_ref, scales_ref, pres_ref,
                    lp_ref, vp_ref, wv_ref, wp_ref, idx_ref,
                    ml_ref, mlp_ref, post_ref):
    b = pl.program_id(0)
    xb = x_ref[0]
    vb = votes_ref[0]
    s = scales_ref[0]
    pr = pres_ref[0]

    P, d = xb.shape
    V = vb.shape[0]

    c_dummy = -2.0 * jnp.log(10.0)
    c_2pi = jnp.log(2.0 * jnp.pi)

    xn = jnp.sum(xb * xb, axis=1, keepdims=True)
    vn = lax.dot_general(jnp.ones((1, d), jnp.float32), vb * vb,
                         (((1,), (1,)), ((), ())),
                         preferred_element_type=jnp.float32,
                         precision=lax.Precision.HIGHEST)
    g = lax.dot_general(xb, vb, (((1,), (1,)), ((), ())),
                        preferred_element_type=jnp.float32,
                        precision=lax.Precision.HIGHEST)
    sq = xn - 2.0 * g + vn

    logs = jnp.log(s)
    inv_s2 = 1.0 / (s * s)
    vlp = -0.5 * sq * inv_s2 - (d * 1.0) * logs - (0.5 * d) * c_2pi

    ml = jnp.log(pr + 1e-16)
    m0 = jnp.maximum(jnp.max(ml), c_dummy)
    lse = m0 + jnp.log(jnp.sum(jnp.exp(ml - m0)) + jnp.exp(c_dummy - m0))
    mlp = ml - lse
    mlp_d = c_dummy - lse

    t = vlp + mlp
    t_d = c_dummy + mlp_d

    mmax = jnp.maximum(jnp.max(t, axis=1, keepdims=True), t_d)
    e = jnp.exp(t - mmax)
    se = jnp.sum(e, axis=1, keepdims=True) + jnp.exp(t_d - mmax)
    point_lp = mmax + jnp.log(se)
    partial = jnp.sum(point_lp)

    post_ref[0] = e / se

    idx = jnp.argmax(t, axis=1).astype(jnp.int32)
    iota_v = lax.broadcasted_iota(jnp.int32, (P, V), 1)
    onehot = (iota_v == idx[:, None]).astype(jnp.float32)
    wv = lax.dot_general(onehot, vb, (((1,), (0,)), ((), ())),
                         preferred_element_type=jnp.float32,
                         precision=lax.Precision.HIGHEST)

    wv_ref[0] = wv
    wp_ref[0, 0, :] = jnp.sum(onehot * pr, axis=1)
    idx_ref[0, 0, :] = idx
    ml_ref[0] = ml
    vp_ref[0] = (ml > c_dummy).astype(jnp.float32)
    mlp_ref[0] = jnp.concatenate(
        [mlp, jnp.full((1, 128), mlp_d, jnp.float32)], axis=1)

    @pl.when(b == 0)
    def _():
        lp_ref[0, 0] = partial

    @pl.when(b != 0)
    def _():
        lp_ref[0, 0] = lp_ref[0, 0] + partial


@jax.jit
def kernel(x, votes, scales, vote_presence_prob):
    B, P, d = x.shape
    V = votes.shape[1]
    f32 = jnp.float32

    out_shapes = (
        jax.ShapeDtypeStruct((1, 1), f32),
        jax.ShapeDtypeStruct((B, 1, V), f32),
        jax.ShapeDtypeStruct((B, P, d), f32),
        jax.ShapeDtypeStruct((B, 1, P), f32),
        jax.ShapeDtypeStruct((B, 1, P), jnp.int32),
        jax.ShapeDtypeStruct((B, 1, V), f32),
        jax.ShapeDtypeStruct((B, 1, V + 128), f32),
        jax.ShapeDtypeStruct((B, P, V), f32),
    )
    grid = (B,)
    outs = pl.pallas_call(
        _capsule_kernel,
        grid=grid,
        in_specs=[
            pl.BlockSpec((1, P, d), lambda b: (b, 0, 0)),
            pl.BlockSpec((1, V, d), lambda b: (b, 0, 0)),
            pl.BlockSpec((1, 1, V), lambda b: (b, 0, 0)),
            pl.BlockSpec((1, 1, V), lambda b: (b, 0, 0)),
        ],
        out_specs=(
            pl.BlockSpec((1, 1), lambda b: (0, 0), memory_space=pltpu.SMEM),
            pl.BlockSpec((1, 1, V), lambda b: (b, 0, 0)),
            pl.BlockSpec((1, P, d), lambda b: (b, 0, 0)),
            pl.BlockSpec((1, 1, P), lambda b: (b, 0, 0)),
            pl.BlockSpec((1, 1, P), lambda b: (b, 0, 0)),
            pl.BlockSpec((1, 1, V), lambda b: (b, 0, 0)),
            pl.BlockSpec((1, 1, V + 128), lambda b: (b, 0, 0)),
            pl.BlockSpec((1, P, V), lambda b: (b, 0, 0)),
        ),
        out_shape=out_shapes,
        compiler_params=pltpu.CompilerParams(
            dimension_semantics=("arbitrary",),
        ),
    )(x, votes.reshape(B, V, d), scales.reshape(B, 1, V),
      vote_presence_prob.reshape(B, 1, V))

    (lp, vote_presence, winning_vote, winning_pres, idx,
     ml_v, mlp_pack, posterior) = outs
    vote_presence = vote_presence.reshape(B, V)
    winning_pres = winning_pres.reshape(B, P)
    idx = idx.reshape(B, P)
    ml_v = ml_v.reshape(B, V)
    mlp_pack = mlp_pack.reshape(B, V + 128)

    c_dummy = jnp.full((B, 1), -2.0 * jnp.log(10.0), f32)
    mixing_logits = jnp.concatenate([ml_v, c_dummy], axis=1)
    mixing_log_prob = jnp.concatenate(
        [mlp_pack[:, :V], mlp_pack[:, V:V + 1]], axis=1)
    mixture_log_prob_per_batch = lp[0, 0]
    is_from_capsule = idx // V
    soft_winner = jnp.zeros_like(winning_vote)
    soft_winner_pres = jnp.zeros_like(winning_pres)
    return (mixture_log_prob_per_batch, vote_presence, winning_vote,
            winning_pres, is_from_capsule, mixing_logits, mixing_log_prob,
            soft_winner, soft_winner_pres, posterior)


# speedup vs baseline: 6.4587x; 1.1752x over previous
import jax
import jax.numpy as jnp
from jax import lax
from jax.experimental import pallas as pl
from jax.experimental.pallas import tpu as pltpu

_BSUB = 4


def _capsule_kernel(x_ref, votes_ref, scales_ref, pres_ref,
                    lp_ref, vp_ref, wv_ref, wp_ref, idx_ref,
                    ml_ref, mlp_ref, post_ref):
    step = pl.program_id(0)
    P, d = x_ref.shape[1], x_ref.shape[2]
    V = votes_ref.shape[1]
    f32 = jnp.float32

    c_dummy = -2.0 * jnp.log(10.0)
    c_2pi = jnp.log(2.0 * jnp.pi)
    ones_col = jnp.ones((V, 1), f32)
    ut = (lax.broadcasted_iota(jnp.int32, (V, V), 0)
          < lax.broadcasted_iota(jnp.int32, (V, V), 1)).astype(f32)
    iota_col = lax.broadcasted_iota(jnp.int32, (V, 1), 0).astype(f32)

    def dot(a, bm, prec=lax.Precision.DEFAULT):
        return lax.dot_general(a, bm, (((1,), (0,)), ((), ())),
                               preferred_element_type=f32, precision=prec)

    partial = jnp.float32(0.0)
    for i in range(_BSUB):
        xb = x_ref[i]
        vb = votes_ref[i]
        s = scales_ref[i]
        pr = pres_ref[i]

        xn = jnp.sum(xb * xb, axis=1, keepdims=True)
        vn = lax.dot_general(jnp.ones((1, d), f32), vb * vb,
                             (((1,), (1,)), ((), ())),
                             preferred_element_type=f32,
                             precision=lax.Precision.HIGHEST)
        g = lax.dot_general(xb, vb, (((1,), (1,)), ((), ())),
                            preferred_element_type=f32,
                            precision=lax.Precision.HIGHEST)

        ml = jnp.log(pr + 1e-16)
        m0 = jnp.maximum(jnp.max(ml), c_dummy)
        lse = m0 + jnp.log(jnp.sum(jnp.exp(ml - m0)) + jnp.exp(c_dummy - m0))
        mlp = ml - lse
        mlp_d = c_dummy - lse

        logs = jnp.log(s)
        arow = -0.5 / (s * s)
        crow = mlp - (d * 1.0) * logs - (0.5 * d) * c_2pi
        t = (xn - 2.0 * g + vn) * arow + crow
        t_d = c_dummy + mlp_d

        mmax_v = jnp.max(t, axis=1, keepdims=True)
        mmax = jnp.maximum(mmax_v, t_d)
        e = jnp.exp(t - mmax)
        se = dot(e, ones_col) + jnp.exp(t_d - mmax)
        point_lp = mmax + jnp.log(se)
        partial = partial + jnp.sum(point_lp)

        post_ref[i] = e * (1.0 / se)

        eq = (t == mmax_v).astype(f32)
        cnt = dot(eq, ut)
        onehot = eq * (cnt == 0.0).astype(f32)
        idx_col = dot(onehot, iota_col)
        vb_hi = vb.astype(jnp.bfloat16).astype(f32)
        wv = dot(onehot, vb_hi) + dot(onehot, vb - vb_hi)
        q = onehot * pr
        q_hi = q.astype(jnp.bfloat16).astype(f32)
        wp_col = dot(q_hi, ones_col) + dot(q - q_hi, ones_col)

        wv_ref[i] = wv
        wp_ref[i] = wp_col
        idx_ref[i] = idx_col.astype(jnp.int32)
        ml_ref[i] = ml
        vp_ref[i] = (ml > c_dummy).astype(f32)
        mlp_ref[i] = jnp.concatenate(
            [mlp, jnp.full((1, 128), mlp_d, f32)], axis=1)

    @pl.when(step == 0)
    def _():
        lp_ref[0, 0] = partial

    @pl.when(step != 0)
    def _():
        lp_ref[0, 0] = lp_ref[0, 0] + partial


@jax.jit
def kernel(x, votes, scales, vote_presence_prob):
    B, P, d = x.shape
    V = votes.shape[1]
    f32 = jnp.float32
    nb = _BSUB

    out_shapes = (
        jax.ShapeDtypeStruct((1, 1), f32),
        jax.ShapeDtypeStruct((B, 1, V), f32),
        jax.ShapeDtypeStruct((B, P, d), f32),
        jax.ShapeDtypeStruct((B, P, 1), f32),
        jax.ShapeDtypeStruct((B, P, 1), jnp.int32),
        jax.ShapeDtypeStruct((B, 1, V), f32),
        jax.ShapeDtypeStruct((B, 1, V + 128), f32),
        jax.ShapeDtypeStruct((B, P, V), f32),
    )
    grid = (B // nb,)
    outs = pl.pallas_call(
        _capsule_kernel,
        grid=grid,
        in_specs=[
            pl.BlockSpec((nb, P, d), lambda b: (b, 0, 0)),
            pl.BlockSpec((nb, V, d), lambda b: (b, 0, 0)),
            pl.BlockSpec((nb, 1, V), lambda b: (b, 0, 0)),
            pl.BlockSpec((nb, 1, V), lambda b: (b, 0, 0)),
        ],
        out_specs=(
            pl.BlockSpec((1, 1), lambda b: (0, 0), memory_space=pltpu.SMEM),
            pl.BlockSpec((nb, 1, V), lambda b: (b, 0, 0)),
            pl.BlockSpec((nb, P, d), lambda b: (b, 0, 0)),
            pl.BlockSpec((nb, P, 1), lambda b: (b, 0, 0)),
            pl.BlockSpec((nb, P, 1), lambda b: (b, 0, 0)),
            pl.BlockSpec((nb, 1, V), lambda b: (b, 0, 0)),
            pl.BlockSpec((nb, 1, V + 128), lambda b: (b, 0, 0)),
            pl.BlockSpec((nb, P, V), lambda b: (b, 0, 0)),
        ),
        out_shape=out_shapes,
        compiler_params=pltpu.CompilerParams(
            dimension_semantics=("arbitrary",),
        ),
    )(x, votes.reshape(B, V, d), scales.reshape(B, 1, V),
      vote_presence_prob.reshape(B, 1, V))

    (lp, vote_presence, winning_vote, winning_pres, idx,
     ml_v, mlp_pack, posterior) = outs
    vote_presence = vote_presence.reshape(B, V)
    winning_pres = winning_pres.reshape(B, P)
    idx = idx.reshape(B, P)
    ml_v = ml_v.reshape(B, V)
    mlp_pack = mlp_pack.reshape(B, V + 128)

    c_dummy = jnp.full((B, 1), -2.0 * jnp.log(10.0), f32)
    mixing_logits = jnp.concatenate([ml_v, c_dummy], axis=1)
    mixing_log_prob = jnp.concatenate(
        [mlp_pack[:, :V], mlp_pack[:, V:V + 1]], axis=1)
    mixture_log_prob_per_batch = lp[0, 0]
    is_from_capsule = idx // V
    soft_winner = jnp.zeros_like(winning_vote)
    soft_winner_pres = jnp.zeros_like(winning_pres)
    return (mixture_log_prob_per_batch, vote_presence, winning_vote,
            winning_pres, is_from_capsule, mixing_logits, mixing_log_prob,
            soft_winner, soft_winner_pres, posterior)
